# baseline (device time: 140762 ns/iter reference)
import jax
import jax.numpy as jnp
from jax import lax
from jax.experimental import pallas as pl
from jax.experimental.pallas import tpu as pltpu

N_DEV = 16
N_EXPERTS = 64
CAP = 25
CAP_PAD = 32
E_LOCAL = N_EXPERTS // N_DEV
BLOCK = E_LOCAL * CAP_PAD
SENTINEL = 4096


def _moe_body(slot_col_ref, slot_row_ref, x_ref, w_ref, out_ref, comm_ref,
              chunk_ref, send1, recv1, send2, recv2):
    n_tok = slot_col_ref.shape[0]
    my = lax.axis_index("i")

    g_iota = lax.broadcasted_iota(jnp.int32, (BLOCK, n_tok), 0)
    G = (slot_row_ref[...] == my * BLOCK + g_iota).astype(jnp.float32)
    xg = jnp.dot(G, x_ref[...], preferred_element_type=jnp.float32)
    compact = jnp.concatenate(
        [
            jnp.dot(
                xg[e * CAP_PAD:(e + 1) * CAP_PAD, :],
                w_ref[e],
                preferred_element_type=jnp.float32,
            )
            for e in range(E_LOCAL)
        ],
        axis=0,
    ).astype(jnp.bfloat16)
    comm_ref[pl.ds(my, 1)] = compact[None, :, :]

    def a2a_send(buf_ref, send_sems, recv_sems):
        sends = []
        for j in range(1, N_DEV):
            peer = lax.rem(my + j, N_DEV)
            rdma = pltpu.make_async_remote_copy(
                src_ref=buf_ref.at[my],
                dst_ref=buf_ref.at[my],
                send_sem=send_sems.at[j - 1],
                recv_sem=recv_sems.at[j - 1],
                device_id=(peer,),
                device_id_type=pl.DeviceIdType.MESH,
            )
            rdma.start()
            sends.append(rdma)
        return sends

    def a2a_wait_recv(buf_ref, send_sems, recv_sems):
        for j in range(1, N_DEV):
            origin = lax.rem(my - j + N_DEV, N_DEV)
            rdma = pltpu.make_async_remote_copy(
                src_ref=buf_ref.at[origin],
                dst_ref=buf_ref.at[origin],
                send_sem=send_sems.at[j - 1],
                recv_sem=recv_sems.at[j - 1],
                device_id=(origin,),
                device_id_type=pl.DeviceIdType.MESH,
            )
            rdma.wait_recv()

    sends1 = a2a_send(comm_ref, send1, recv1)
    a2a_wait_recv(comm_ref, send1, recv1)

    my_slots = slot_col_ref[pl.ds(my * BLOCK, BLOCK)]
    p_iota = lax.broadcasted_iota(jnp.int32, (BLOCK, N_DEV * BLOCK), 1)
    P = (my_slots == p_iota).astype(jnp.bfloat16)
    comm_all = comm_ref[...].reshape(N_DEV * BLOCK, -1)
    chunk = jnp.dot(P, comm_all, preferred_element_type=jnp.float32)
    chunk_ref[pl.ds(my, 1)] = chunk.astype(jnp.bfloat16)[None, :, :]

    sends2 = a2a_send(chunk_ref, send2, recv2)
    for rdma in sends1:
        rdma.wait_send()
    a2a_wait_recv(chunk_ref, send2, recv2)
    for rdma in sends2:
        rdma.wait_send()

    out_ref[...] = chunk_ref[...].reshape(n_tok, -1).astype(jnp.float32)


def kernel(x, router_W, route_idx, expert_W):
    del router_W
    n_tok, d = x.shape
    h = expert_W.shape[-1]

    e = route_idx[:, 0].astype(jnp.int32)

    onehot = (e[:, None] == jnp.arange(N_EXPERTS, dtype=jnp.int32)[None, :]).astype(
        jnp.int32
    )
    before = jnp.cumsum(onehot, axis=0) - onehot
    rank = jnp.sum(before * onehot, axis=1)
    accepted = rank < CAP

    slot = jnp.where(accepted, e * CAP_PAD + rank, SENTINEL)

    return pl.pallas_call(
        _moe_body,
        out_shape=jax.ShapeDtypeStruct((n_tok, h), jnp.float32),
        in_specs=[pl.BlockSpec(memory_space=pltpu.VMEM)] * 4,
        out_specs=pl.BlockSpec(memory_space=pltpu.VMEM),
        scratch_shapes=[
            pltpu.VMEM((N_DEV, BLOCK, h), jnp.bfloat16),
            pltpu.VMEM((N_DEV, BLOCK, h), jnp.bfloat16),
            pltpu.SemaphoreType.DMA((N_DEV - 1,)),
            pltpu.SemaphoreType.DMA((N_DEV - 1,)),
            pltpu.SemaphoreType.DMA((N_DEV - 1,)),
            pltpu.SemaphoreType.DMA((N_DEV - 1,)),
        ],
    )(slot[:, None], slot[None, :], x, expert_W)


# device time: 85792 ns/iter; 1.6407x vs baseline; 1.6407x over previous
import jax
import jax.numpy as jnp
from jax import lax
from jax.experimental import pallas as pl
from jax.experimental.pallas import tpu as pltpu

N_DEV = 16
N_EXPERTS = 64
CAP = 25
CAP_PAD = 32
E_LOCAL = N_EXPERTS // N_DEV
BLOCK = E_LOCAL * CAP_PAD
SENTINEL = 4096
N_R = N_DEV // 2 - 1
N_L = N_DEV // 2


def _moe_body(slot_col_ref, slot_row_ref, x_ref, w_ref, out_ref, comm_ref,
              send_r, recv_r, send_l, recv_l):
    n_tok = slot_col_ref.shape[0]
    my = lax.axis_index("i")
    right = lax.rem(my + 1, N_DEV)
    left = lax.rem(my - 1 + N_DEV, N_DEV)

    g_iota = lax.broadcasted_iota(jnp.int32, (BLOCK, n_tok), 0)
    G = (slot_row_ref[...] == my * BLOCK + g_iota).astype(jnp.float32)
    xg = jnp.dot(G, x_ref[...], preferred_element_type=jnp.float32)
    compact = jnp.concatenate(
        [
            jnp.dot(
                xg[e * CAP_PAD:(e + 1) * CAP_PAD, :],
                w_ref[e],
                preferred_element_type=jnp.float32,
            )
            for e in range(E_LOCAL)
        ],
        axis=0,
    ).astype(jnp.bfloat16)
    comm_ref[pl.ds(my, 1)] = compact[None, :, :]

    p_iota = lax.broadcasted_iota(jnp.int32, (n_tok, BLOCK), 1)

    def scatter_matmul(origin, block):
        P = (slot_col_ref[...] == origin * BLOCK + p_iota).astype(jnp.bfloat16)
        return jnp.dot(P, block, preferred_element_type=jnp.float32)

    def load_block(origin):
        return comm_ref[pl.ds(origin, 1)].reshape(BLOCK, -1)

    for k in range(N_L):
        o_r = lax.rem(my - k + N_DEV, N_DEV)
        o_l = lax.rem(my + k, N_DEV)
        rdma_r = None
        if k < N_R:
            rdma_r = pltpu.make_async_remote_copy(
                src_ref=comm_ref.at[o_r],
                dst_ref=comm_ref.at[o_r],
                send_sem=send_r.at[k],
                recv_sem=recv_r.at[k],
                device_id=(right,),
                device_id_type=pl.DeviceIdType.MESH,
            )
            rdma_r.start()
        rdma_l = pltpu.make_async_remote_copy(
            src_ref=comm_ref.at[o_l],
            dst_ref=comm_ref.at[o_l],
            send_sem=send_l.at[k],
            recv_sem=recv_l.at[k],
            device_id=(left,),
            device_id_type=pl.DeviceIdType.MESH,
        )
        rdma_l.start()

        if k == 0:
            out_ref[...] = scatter_matmul(my, compact)
        else:
            targets = jnp.concatenate(
                [o_r * BLOCK + p_iota, o_l * BLOCK + p_iota], axis=1
            )
            P2 = (slot_col_ref[...] == targets).astype(jnp.bfloat16)
            B2 = jnp.concatenate([load_block(o_r), load_block(o_l)], axis=0)
            out_ref[...] += jnp.dot(P2, B2, preferred_element_type=jnp.float32)

        if rdma_r is not None:
            rdma_r.wait()
        rdma_l.wait()

    o_last = lax.rem(my + N_L, N_DEV)
    out_ref[...] += scatter_matmul(o_last, load_block(o_last))


def kernel(x, router_W, route_idx, expert_W):
    del router_W
    n_tok, d = x.shape
    h = expert_W.shape[-1]

    e = route_idx[:, 0].astype(jnp.int32)

    onehot = (e[:, None] == jnp.arange(N_EXPERTS, dtype=jnp.int32)[None, :]).astype(
        jnp.int32
    )
    before = jnp.cumsum(onehot, axis=0) - onehot
    rank = jnp.sum(before * onehot, axis=1)
    accepted = rank < CAP

    slot = jnp.where(accepted, e * CAP_PAD + rank, SENTINEL)

    return pl.pallas_call(
        _moe_body,
        out_shape=jax.ShapeDtypeStruct((n_tok, h), jnp.float32),
        in_specs=[pl.BlockSpec(memory_space=pltpu.VMEM)] * 4,
        out_specs=pl.BlockSpec(memory_space=pltpu.VMEM),
        scratch_shapes=[
            pltpu.VMEM((N_DEV, BLOCK, h), jnp.bfloat16),
            pltpu.SemaphoreType.DMA((N_R,)),
            pltpu.SemaphoreType.DMA((N_R,)),
            pltpu.SemaphoreType.DMA((N_L,)),
            pltpu.SemaphoreType.DMA((N_L,)),
        ],
    )(slot[:, None], slot[None, :], x, expert_W)


# device time: 74580 ns/iter; 1.8874x vs baseline; 1.1503x over previous
import jax
import jax.numpy as jnp
from jax import lax
from jax.experimental import pallas as pl
from jax.experimental.pallas import tpu as pltpu

N_DEV = 16
N_EXPERTS = 64
CAP = 25
CAP_PAD = 32
E_LOCAL = N_EXPERTS // N_DEV
BLOCK = E_LOCAL * CAP_PAD
SENTINEL = 4096
N_R = N_DEV // 2 - 1
N_L = N_DEV // 2


def _moe_body(e_ref, x_ref, w_ref, out_ref, comm_ref,
              send_r, recv_r, send_l, recv_l):
    n_tok = e_ref.shape[0]
    my = lax.axis_index("i")
    right = lax.rem(my + 1, N_DEV)
    left = lax.rem(my - 1 + N_DEV, N_DEV)

    e_col = e_ref[...]
    onehot = (
        e_col == lax.broadcasted_iota(jnp.int32, (n_tok, N_EXPERTS), 1)
    ).astype(jnp.bfloat16)
    row_i = lax.broadcasted_iota(jnp.int32, (n_tok, n_tok), 0)
    col_i = lax.broadcasted_iota(jnp.int32, (n_tok, n_tok), 1)
    L = (col_i < row_i).astype(jnp.bfloat16)
    cnt = jnp.dot(L, onehot, preferred_element_type=jnp.float32)
    rank = jnp.sum(
        cnt * onehot.astype(jnp.float32), axis=1, keepdims=True
    ).astype(jnp.int32)
    slot = jnp.where(rank < CAP, e_col * CAP_PAD + rank, SENTINEL)

    p_iota = lax.broadcasted_iota(jnp.int32, (n_tok, BLOCK), 1)
    P_my = (slot == my * BLOCK + p_iota).astype(jnp.float32)
    xg = lax.dot_general(
        P_my,
        x_ref[...],
        dimension_numbers=(((0,), (0,)), ((), ())),
        preferred_element_type=jnp.float32,
    )
    compact = jnp.concatenate(
        [
            jnp.dot(
                xg[e * CAP_PAD:(e + 1) * CAP_PAD, :],
                w_ref[e],
                preferred_element_type=jnp.float32,
            )
            for e in range(E_LOCAL)
        ],
        axis=0,
    ).astype(jnp.bfloat16)
    comm_ref[pl.ds(my, 1)] = compact[None, :, :]

    def scatter_matmul(origin, block):
        P = (slot == origin * BLOCK + p_iota).astype(jnp.bfloat16)
        return jnp.dot(P, block, preferred_element_type=jnp.float32)

    def load_block(origin):
        return comm_ref[pl.ds(origin, 1)].reshape(BLOCK, -1)

    for k in range(N_L):
        o_r = lax.rem(my - k + N_DEV, N_DEV)
        o_l = lax.rem(my + k, N_DEV)
        rdma_r = None
        if k < N_R:
            rdma_r = pltpu.make_async_remote_copy(
                src_ref=comm_ref.at[o_r],
                dst_ref=comm_ref.at[o_r],
                send_sem=send_r.at[k],
                recv_sem=recv_r.at[k],
                device_id=(right,),
                device_id_type=pl.DeviceIdType.MESH,
            )
            rdma_r.start()
        rdma_l = pltpu.make_async_remote_copy(
            src_ref=comm_ref.at[o_l],
            dst_ref=comm_ref.at[o_l],
            send_sem=send_l.at[k],
            recv_sem=recv_l.at[k],
            device_id=(left,),
            device_id_type=pl.DeviceIdType.MESH,
        )
        rdma_l.start()

        if k == 0:
            out_ref[...] = scatter_matmul(my, compact)
        else:
            targets = jnp.concatenate(
                [o_r * BLOCK + p_iota, o_l * BLOCK + p_iota], axis=1
            )
            P2 = (slot == targets).astype(jnp.bfloat16)
            B2 = jnp.concatenate([load_block(o_r), load_block(o_l)], axis=0)
            out_ref[...] += jnp.dot(P2, B2, preferred_element_type=jnp.float32)

        if rdma_r is not None:
            rdma_r.wait()
        rdma_l.wait()

    o_last = lax.rem(my + N_L, N_DEV)
    out_ref[...] += scatter_matmul(o_last, load_block(o_last))


def kernel(x, router_W, route_idx, expert_W):
    del router_W
    n_tok, d = x.shape
    h = expert_W.shape[-1]

    return pl.pallas_call(
        _moe_body,
        out_shape=jax.ShapeDtypeStruct((n_tok, h), jnp.float32),
        in_specs=[pl.BlockSpec(memory_space=pltpu.VMEM)] * 3,
        out_specs=pl.BlockSpec(memory_space=pltpu.VMEM),
        scratch_shapes=[
            pltpu.VMEM((N_DEV, BLOCK, h), jnp.bfloat16),
            pltpu.SemaphoreType.DMA((N_R,)),
            pltpu.SemaphoreType.DMA((N_R,)),
            pltpu.SemaphoreType.DMA((N_L,)),
            pltpu.SemaphoreType.DMA((N_L,)),
        ],
    )(route_idx[:, :1].astype(jnp.int32), x, expert_W)
